# baseline (speedup 1.0000x reference)
import jax
import jax.numpy as jnp
from jax.experimental import pallas as pl
from jax.experimental.pallas import tpu as pltpu

_REPARAM_OFFSET = 2.0 ** -18
_PEDESTAL = _REPARAM_OFFSET ** 2
_BETA_MIN = 1e-6
_BOUND_BETA = float((_BETA_MIN + _PEDESTAL) ** 0.5)
_BOUND_A = float(_PEDESTAL ** 0.5)

_BLK = 2048


def _gdn_kernel(x_ref, beta_ref, a_ref, o_ref):
    a_r = jnp.maximum(a_ref[...], _BOUND_A) ** 2 - _PEDESTAL
    beta_r = jnp.maximum(beta_ref[...], _BOUND_BETA) ** 2 - _PEDESTAL
    m = jnp.dot(a_r, a_r.T, preferred_element_type=jnp.float32)
    x = x_ref[0]
    x2 = x * x
    denom = jnp.dot(m, x2, preferred_element_type=jnp.float32) + beta_r
    o_ref[0] = x * jax.lax.rsqrt(denom)


def kernel(x, beta, A):
    N, C, H, W = x.shape
    hw = H * W
    xr = x.reshape(N, C, hw)
    out = pl.pallas_call(
        _gdn_kernel,
        grid=(N, hw // _BLK),
        in_specs=[
            pl.BlockSpec((1, C, _BLK), lambda n, j: (n, 0, j)),
            pl.BlockSpec((C, 1), lambda n, j: (0, 0)),
            pl.BlockSpec((C, A.shape[1]), lambda n, j: (0, 0)),
        ],
        out_specs=pl.BlockSpec((1, C, _BLK), lambda n, j: (n, 0, j)),
        out_shape=jax.ShapeDtypeStruct((N, C, hw), x.dtype),
        compiler_params=pltpu.CompilerParams(
            dimension_semantics=("parallel", "parallel"),
        ),
    )(xr, beta.reshape(C, 1), A)
    return out.reshape(N, C, H, W)


# speedup vs baseline: 2.3974x; 2.3974x over previous
import jax
import jax.numpy as jnp
import numpy as np
from jax.experimental import pallas as pl
from jax.experimental.pallas import tpu as pltpu

_REPARAM_OFFSET = 2.0 ** -18
_PEDESTAL = _REPARAM_OFFSET ** 2
_BETA_MIN = 1e-6
_BOUND_BETA = float((_BETA_MIN + _PEDESTAL) ** 0.5)
_BOUND_A = float(_PEDESTAL ** 0.5)

_HB = 8


def _gdn_body(x_ref, a1_ref, a2_ref, o_ref, t_ref):
    c8, w = a2_ref.shape[0], x_ref.shape[3]
    r8 = a1_ref.shape[0]
    x = x_ref[0].reshape(c8, w)
    x2 = x * x
    t_ref[0:r8, :] = jnp.dot(a1_ref[...], x2, preferred_element_type=jnp.float32)
    ones_row = jax.lax.broadcasted_iota(jnp.int32, (8, w), 0) == 0
    t_ref[r8:r8 + 8, :] = jnp.where(ones_row, 1.0, 0.0)
    denom = jnp.dot(a2_ref[...], t_ref[...], preferred_element_type=jnp.float32)
    out = x * jax.lax.rsqrt(denom)
    o_ref[0] = out.reshape(c8 // _HB, _HB, w)


def kernel(x, beta, A):
    N, C, H, W = x.shape
    R = A.shape[1]
    r8, c8 = R * 8, C * _HB

    beta_r = jnp.maximum(beta, _BOUND_BETA) ** 2 - _PEDESTAL
    a_r = jnp.maximum(A, _BOUND_A) ** 2 - _PEDESTAL
    eye8 = jnp.eye(8, dtype=jnp.float32)
    big_a1 = jnp.kron(a_r.T, eye8)
    big_a2 = jnp.kron(a_r, eye8)
    beta_col = jnp.repeat(beta_r, 8)[:, None]
    pad = jnp.zeros((c8, 7), jnp.float32)
    big_a2 = jnp.concatenate([big_a2, beta_col, pad], axis=1)

    return pl.pallas_call(
        _gdn_body,
        grid=(N, H // _HB),
        in_specs=[
            pl.BlockSpec((1, C, _HB, W), lambda n, h: (n, 0, h, 0)),
            pl.BlockSpec((r8, c8), lambda n, h: (0, 0)),
            pl.BlockSpec((c8, r8 + 8), lambda n, h: (0, 0)),
        ],
        out_specs=pl.BlockSpec((1, C, _HB, W), lambda n, h: (n, 0, h, 0)),
        out_shape=jax.ShapeDtypeStruct((N, C, H, W), x.dtype),
        scratch_shapes=[pltpu.VMEM((r8 + 8, W), jnp.float32)],
        compiler_params=pltpu.CompilerParams(
            dimension_semantics=("parallel", "parallel"),
        ),
    )(x, big_a1, big_a2)


# speedup vs baseline: 2.9301x; 1.2222x over previous
import jax
import jax.numpy as jnp
import numpy as np
from jax.experimental import pallas as pl
from jax.experimental.pallas import tpu as pltpu

_REPARAM_OFFSET = 2.0 ** -18
_PEDESTAL = _REPARAM_OFFSET ** 2
_BETA_MIN = 1e-6
_BOUND_BETA = float((_BETA_MIN + _PEDESTAL) ** 0.5)
_BOUND_A = float(_PEDESTAL ** 0.5)

_HB = 16


def _gdn_body(x_ref, a1_ref, a2_ref, o_ref, t_ref):
    c8, w = a2_ref.shape[0], x_ref.shape[3]
    r8 = a1_ref.shape[0]
    x = x_ref[0].reshape(c8, w)
    x2 = x * x
    t_ref[0:r8, :] = jnp.dot(a1_ref[...], x2, preferred_element_type=jnp.float32)
    ones_row = jax.lax.broadcasted_iota(jnp.int32, (8, w), 0) == 0
    t_ref[r8:r8 + 8, :] = jnp.where(ones_row, 1.0, 0.0)
    denom = jnp.dot(a2_ref[...], t_ref[...], preferred_element_type=jnp.float32)
    out = x * jax.lax.rsqrt(denom)
    o_ref[0] = out.reshape(c8 // _HB, _HB, w)


def kernel(x, beta, A):
    N, C, H, W = x.shape
    R = A.shape[1]
    r8, c8 = R * _HB, C * _HB

    beta_r = jnp.maximum(beta, _BOUND_BETA) ** 2 - _PEDESTAL
    a_r = jnp.maximum(A, _BOUND_A) ** 2 - _PEDESTAL
    eye_hb = jnp.eye(_HB, dtype=jnp.float32)
    big_a1 = jnp.kron(a_r.T, eye_hb)
    big_a2 = jnp.kron(a_r, eye_hb)
    beta_col = jnp.repeat(beta_r, _HB)[:, None]
    pad = jnp.zeros((c8, 7), jnp.float32)
    big_a2 = jnp.concatenate([big_a2, beta_col, pad], axis=1)

    return pl.pallas_call(
        _gdn_body,
        grid=(N, H // _HB),
        in_specs=[
            pl.BlockSpec((1, C, _HB, W), lambda n, h: (n, 0, h, 0)),
            pl.BlockSpec((r8, c8), lambda n, h: (0, 0)),
            pl.BlockSpec((c8, r8 + 8), lambda n, h: (0, 0)),
        ],
        out_specs=pl.BlockSpec((1, C, _HB, W), lambda n, h: (n, 0, h, 0)),
        out_shape=jax.ShapeDtypeStruct((N, C, H, W), x.dtype),
        scratch_shapes=[pltpu.VMEM((r8 + 8, W), jnp.float32)],
        compiler_params=pltpu.CompilerParams(
            dimension_semantics=("parallel", "parallel"),
        ),
    )(x, big_a1, big_a2)
